# baseline (speedup 1.0000x reference)
import functools

import jax
import jax.numpy as jnp
from jax import lax
from jax.experimental import pallas as pl
from jax.experimental.pallas import tpu as pltpu
from jax.experimental.pallas import tpu_sc as plsc

_HIDDEN = 1024
_NC = 2
_NS = 16
_NW = _NC * _NS
_BATCH = 4
_SEQ = 8192
_WPB = _NW // _BATCH
_BPW = _SEQ // _WPB
_CHUNK = 32
_NCHUNK = _BPW // _CHUNK

_mesh = plsc.VectorSubcoreMesh(core_axis_name="c", subcore_axis_name="s")


@functools.partial(
    pl.kernel,
    mesh=_mesh,
    out_type=jax.ShapeDtypeStruct((_BATCH, _SEQ, _HIDDEN), jnp.float32),
    scratch_types=[
        pltpu.VMEM((_BPW,), jnp.int32),
        pltpu.VMEM((2, _CHUNK, _HIDDEN), jnp.float32),
        pltpu.SemaphoreType.DMA,
        pltpu.SemaphoreType.DMA,
    ],
)
def _gather_kernel(ids_hbm, table_hbm, out_hbm, idx_v, rows_v, gsem, osem):
    wid = lax.axis_index("s") * _NC + lax.axis_index("c")
    row = wid // _WPB
    col = (wid % _WPB) * _BPW
    pltpu.sync_copy(ids_hbm.at[row, pl.ds(col, _BPW)], idx_v)

    def gather(j):
        jc = jnp.minimum(j, _NCHUNK - 1)
        pltpu.async_copy(
            table_hbm.at[idx_v.at[pl.ds(jc * _CHUNK, _CHUNK)]],
            rows_v.at[j % 2],
            gsem,
        )

    gather(0)
    gather(1)

    def body(j, carry):
        buf = rows_v.at[j % 2]
        pltpu.make_async_copy(table_hbm.at[pl.ds(0, _CHUNK)], buf, gsem).wait()
        pltpu.async_copy(
            buf, out_hbm.at[row, pl.ds(col + j * _CHUNK, _CHUNK)], osem
        )
        pltpu.make_async_copy(buf, out_hbm.at[row, pl.ds(col, _CHUNK)], osem).wait()
        gather(j + 2)
        return carry

    lax.fori_loop(0, _NCHUNK, body, 0)

    pltpu.make_async_copy(table_hbm.at[pl.ds(0, _CHUNK)], rows_v.at[0], gsem).wait()
    pltpu.make_async_copy(table_hbm.at[pl.ds(0, _CHUNK)], rows_v.at[1], gsem).wait()


def kernel(input_ids, table):
    return _gather_kernel(input_ids, table)


# speedup vs baseline: 1.0100x; 1.0100x over previous
import functools

import jax
import jax.numpy as jnp
from jax import lax
from jax.experimental import pallas as pl
from jax.experimental.pallas import tpu as pltpu
from jax.experimental.pallas import tpu_sc as plsc

_HIDDEN = 1024
_NC = 2
_NS = 16
_NW = _NC * _NS
_BATCH = 4
_SEQ = 8192
_WPB = _NW // _BATCH
_BPW = _SEQ // _WPB
_CHUNK = 16
_NCHUNK = _BPW // _CHUNK
_NSLOT = 4

_mesh = plsc.VectorSubcoreMesh(core_axis_name="c", subcore_axis_name="s")


@functools.partial(
    pl.kernel,
    mesh=_mesh,
    out_type=jax.ShapeDtypeStruct((_BATCH, _SEQ, _HIDDEN), jnp.float32),
    scratch_types=[
        pltpu.VMEM((_BPW,), jnp.int32),
        pltpu.VMEM((_NSLOT, _CHUNK, _HIDDEN), jnp.float32),
        pltpu.SemaphoreType.DMA,
        pltpu.SemaphoreType.DMA,
        pltpu.SemaphoreType.DMA,
        pltpu.SemaphoreType.DMA,
        pltpu.SemaphoreType.DMA,
        pltpu.SemaphoreType.DMA,
        pltpu.SemaphoreType.DMA,
        pltpu.SemaphoreType.DMA,
    ],
)
def _gather_kernel(ids_hbm, table_hbm, out_hbm, idx_v, rows_v, *sems):
    gsem = sems[:_NSLOT]
    osem = sems[_NSLOT:]
    wid = lax.axis_index("s") * _NC + lax.axis_index("c")
    row = wid // _WPB
    col = (wid % _WPB) * _BPW
    pltpu.sync_copy(ids_hbm.at[row, pl.ds(col, _BPW)], idx_v)

    def gather(j, b):
        jc = jnp.minimum(j, _NCHUNK - 1)
        pltpu.async_copy(
            table_hbm.at[idx_v.at[pl.ds(jc * _CHUNK, _CHUNK)]],
            rows_v.at[b],
            gsem[b],
        )

    def out(j, b):
        pltpu.async_copy(
            rows_v.at[b], out_hbm.at[row, pl.ds(col + j * _CHUNK, _CHUNK)], osem[b]
        )

    def wait_g(b):
        pltpu.make_async_copy(
            table_hbm.at[pl.ds(0, _CHUNK)], rows_v.at[b], gsem[b]
        ).wait()

    def wait_o(b):
        pltpu.make_async_copy(
            rows_v.at[b], out_hbm.at[row, pl.ds(col, _CHUNK)], osem[b]
        ).wait()

    gather(0, 0)
    gather(1, 1)
    for j in range(2):
        wait_g(j)
        out(j, j)
        gather(j + 2, (j + 2) % _NSLOT)
    for j in range(2, 4):
        wait_g(j)
        out(j, j)
        wait_o((j + 2) % _NSLOT)
        gather(j + 2, (j + 2) % _NSLOT)

    def body(i, carry):
        for b in range(_NSLOT):
            j = _NSLOT * i + b
            wait_g(b)
            out(j, b)
            wait_o((b + 2) % _NSLOT)
            gather(j + 2, (b + 2) % _NSLOT)
        return carry

    lax.fori_loop(1, _NCHUNK // _NSLOT, body, 0)

    wait_g(0)
    wait_g(1)
    wait_o(2)
    wait_o(3)


def kernel(input_ids, table):
    return _gather_kernel(input_ids, table)


# speedup vs baseline: 1.0293x; 1.0191x over previous
import functools

import jax
import jax.numpy as jnp
from jax import lax
from jax.experimental import pallas as pl
from jax.experimental.pallas import tpu as pltpu
from jax.experimental.pallas import tpu_sc as plsc

_HIDDEN = 1024
_NC = 2
_NS = 16
_NW = _NC * _NS
_BATCH = 4
_SEQ = 8192
_WPB = _NW // _BATCH
_BPW = _SEQ // _WPB
_CHUNK = 16
_NCHUNK = _BPW // _CHUNK
_NSLOT = 4
_NSP = 2

_mesh = plsc.VectorSubcoreMesh(core_axis_name="c", subcore_axis_name="s")


@functools.partial(
    pl.kernel,
    mesh=_mesh,
    out_type=jax.ShapeDtypeStruct((_BATCH, _SEQ, _HIDDEN), jnp.float32),
    scratch_types=[
        pltpu.VMEM((_BPW,), jnp.int32),
        pltpu.VMEM((_NSLOT, _CHUNK, _HIDDEN), jnp.float32),
        pltpu.VMEM_SHARED((_NS, _NSP, _CHUNK, _HIDDEN), jnp.float32),
    ]
    + [pltpu.SemaphoreType.DMA] * (_NSLOT + _NSLOT + _NSP),
)
def _gather_kernel(ids_hbm, table_hbm, out_hbm, idx_v, rows_v, spmem, *sems):
    gsem = sems[:_NSLOT]
    csem = sems[_NSLOT : 2 * _NSLOT]
    dsem = sems[2 * _NSLOT :]
    wid = lax.axis_index("s") * _NC + lax.axis_index("c")
    sid = lax.axis_index("s")
    row = wid // _WPB
    col = (wid % _WPB) * _BPW
    pltpu.sync_copy(ids_hbm.at[row, pl.ds(col, _BPW)], idx_v)

    def gather(j, b):
        jc = jnp.minimum(j, _NCHUNK - 1)
        pltpu.async_copy(
            table_hbm.at[idx_v.at[pl.ds(jc * _CHUNK, _CHUNK)]],
            rows_v.at[b],
            gsem[b],
        )

    def xbar(b, q):
        pltpu.async_copy(rows_v.at[b], spmem.at[sid, q], csem[b])

    def dma(j, q):
        pltpu.async_copy(
            spmem.at[sid, q],
            out_hbm.at[row, pl.ds(col + j * _CHUNK, _CHUNK)],
            dsem[q],
        )

    def wait_g(b):
        pltpu.make_async_copy(
            table_hbm.at[pl.ds(0, _CHUNK)], rows_v.at[b], gsem[b]
        ).wait()

    def wait_c(b):
        pltpu.make_async_copy(rows_v.at[b], spmem.at[sid, 0], csem[b]).wait()

    def wait_d(q):
        pltpu.make_async_copy(
            spmem.at[sid, q], out_hbm.at[row, pl.ds(col, _CHUNK)], dsem[q]
        ).wait()

    def step(j, b, first):
        q = b % _NSP
        wait_g(b)
        if not first:
            wait_d(q)
        xbar(b, q)
        gather(j + 2, (b + 2) % _NSLOT)
        wait_c(b)
        dma(j, q)

    gather(0, 0)
    gather(1, 1)
    for j in range(_NSLOT):
        step(j, j, first=j < _NSP)

    def body(i, carry):
        for b in range(_NSLOT):
            step(_NSLOT * i + b, b, first=False)
        return carry

    lax.fori_loop(1, _NCHUNK // _NSLOT, body, 0)

    wait_g(0)
    wait_g(1)
    wait_d(0)
    wait_d(1)


def kernel(input_ids, table):
    return _gather_kernel(input_ids, table)
